# Initial kernel scaffold; baseline (speedup 1.0000x reference)
import jax
import jax.numpy as jnp
from jax.experimental import pallas as pl


def kernel(f_BChw, embedding, phi_w, phi_b):
    raise NotImplementedError("write your pallas kernel here")


# speedup vs baseline: 1.1311x; 1.1311x over previous
import functools

import jax
import jax.numpy as jnp
import numpy as np
from jax import lax
from jax.experimental import pallas as pl
from jax.experimental.pallas import tpu as pltpu
from jax.experimental.pallas import tpu_sc as plsc

VOCAB = 8192
CVAE = 64
BETA = 0.25
PNS = (1, 2, 4, 8, 16, 32)
RESI = 0.5
NPHI = 4
B = 16
H = W = 32
HW = H * W
NTOK = B * HW

PREC_SCORES = lax.Precision.DEFAULT
PREC_CONV = lax.Precision.DEFAULT
PREC_LIN = lax.Precision.HIGHEST


def _phi_k_map():
    ticks = np.linspace(1.0 / 3.0 / NPHI, 1.0 - 1.0 / 3.0 / NPHI, NPHI)
    return [int(np.argmin(np.abs(ticks - (si / (len(PNS) - 1)))))
            for si in range(len(PNS))]


K_MAP = _phi_k_map()


def _cubic_resize_mat(in_size, out_size):
    inv_scale = in_size / out_size
    sample_f = (np.arange(out_size) + 0.5) * inv_scale - 0.5
    x = np.abs(sample_f[np.newaxis, :] - np.arange(in_size)[:, np.newaxis])

    def keys(x):
        out = ((1.5 * x - 2.5) * x * x + 1.0) * (x <= 1.0)
        out = out + (((-0.5 * x + 2.5) * x - 4.0) * x + 2.0) * ((x > 1.0) & (x < 2.0))
        return out

    w = keys(x)
    tot = w.sum(axis=0, keepdims=True)
    w = np.where(np.abs(tot) > 1000 * np.finfo(np.float32).eps, w / tot, 0.0)
    w = np.where(((sample_f >= -0.5) & (sample_f <= in_size - 0.5))[np.newaxis, :],
                 w, 0.0)
    return w.T


def _up2_mat(pn):
    u = _cubic_resize_mat(pn, H)
    u2 = np.einsum('yp,xq->yxpq', u, u).reshape(HW, pn * pn)
    return u2.astype(np.float32)


def _down2_mat(pn):
    k = H // pn
    d = np.zeros((pn, H))
    for i in range(pn):
        d[i, i * k:(i + 1) * k] = 1.0 / k
    d2 = np.einsum('ph,qw->pqhw', d, d).reshape(pn * pn, HW)
    return d2.astype(np.float32)


UP2 = {pn: _up2_mat(pn) for pn in PNS[:-1]}
DOWN2 = {pn: _down2_mat(pn) for pn in PNS[:-1]}


def _argmax_body(si, frest_ref, emb_ref, d2_ref, idx_ref, rest_scr, embn_scr):
    pn = PNS[si]
    n = B * pn * pn

    for c in range(8):
        e = emb_ref[pl.ds(c * 1024, 1024), :]
        nrm = jnp.sqrt(jnp.sum(e * e, axis=1, keepdims=True))
        embn_scr[pl.ds(c * 1024, 1024), :] = e / jnp.clip(nrm, 1e-12, None)

    if si != len(PNS) - 1:
        for b in range(B):
            blk = lax.dot_general(
                d2_ref[...], frest_ref[pl.ds(b * HW, HW), :],
                (((1,), (0,)), ((), ())), precision=PREC_LIN,
                preferred_element_type=jnp.float32)
            rest_scr[pl.ds(b * pn * pn, pn * pn), :] = blk
        src = rest_scr
    else:
        src = frest_ref

    cs = min(256, n)
    vt = 2048

    def chunk(base):
        x = src[pl.ds(base, cs), :]
        nrm = jnp.sqrt(jnp.sum(x * x, axis=1, keepdims=True))
        xn = x / jnp.clip(nrm, 1e-12, None)
        best = jnp.full((cs, 1), -jnp.inf, jnp.float32)
        bidx = jnp.zeros((cs, 1), jnp.int32)
        for v in range(VOCAB // vt):
            s = lax.dot_general(xn, embn_scr[pl.ds(v * vt, vt), :],
                                (((1,), (1,)), ((), ())),
                                precision=PREC_SCORES,
                                preferred_element_type=jnp.float32)
            mt = jnp.max(s, axis=1, keepdims=True)
            iota = lax.broadcasted_iota(jnp.int32, (cs, vt), 1) + v * vt
            it = jnp.min(jnp.where(s == mt, iota, VOCAB), axis=1,
                         keepdims=True)
            upd = mt > best
            bidx = jnp.where(upd, it, bidx)
            best = jnp.where(upd, mt, best)
        idx_ref[pl.ds(base, cs), :] = bidx

    nchunks = n // cs
    if nchunks <= 4:
        for i in range(nchunks):
            chunk(i * cs)
    else:
        def body(i, carry):
            chunk(i * cs)
            return carry
        lax.fori_loop(0, nchunks, body, 0)


def _make_argmax_call(si):
    pn = PNS[si]
    n = B * pn * pn
    nrest = 8 if si == len(PNS) - 1 else max(n, 8)
    scratch = [pltpu.VMEM((nrest, CVAE), jnp.float32),
               pltpu.VMEM((VOCAB, CVAE), jnp.float32)]
    out_shape = jax.ShapeDtypeStruct((n, 1), jnp.int32)
    body = functools.partial(_argmax_body, si)
    if si != len(PNS) - 1:
        d2 = jnp.asarray(DOWN2[pn])
        call = pl.pallas_call(body, out_shape=out_shape, scratch_shapes=scratch)
        return lambda frest, emb: call(frest, emb, d2)
    else:
        def body5(frest_ref, emb_ref, idx_ref, rest_scr, embn_scr):
            _argmax_body(si, frest_ref, emb_ref, None, idx_ref, rest_scr,
                         embn_scr)
        call = pl.pallas_call(body5, out_shape=out_shape,
                              scratch_shapes=scratch)
        return lambda frest, emb: call(frest, emb)


def _make_gather_call(n):
    nc, ns = 2, 16
    nw = nc * ns
    rw = max(8, n // nw)
    nworkers = n // rw
    cs = min(rw, 128)
    nchunks = rw // cs
    mesh = plsc.VectorSubcoreMesh(core_axis_name="c", subcore_axis_name="s",
                                  num_cores=nc, num_subcores=ns)

    @functools.partial(
        pl.kernel, mesh=mesh,
        out_type=jax.ShapeDtypeStruct((n, 128), jnp.float32),
        scratch_types=[
            pltpu.VMEM((cs,), jnp.int32),
            pltpu.VMEM((cs, 128), jnp.float32),
            pltpu.SemaphoreType.DMA,
        ],
    )
    def gk(table_hbm, idx_hbm, out_hbm, idx_v, rows_v, sem):
        wid = lax.axis_index("s") * nc + lax.axis_index("c")

        @pl.when(wid < nworkers)
        def _():
            for j in range(nchunks):
                base = wid * rw + j * cs
                pltpu.sync_copy(idx_hbm.at[pl.ds(base, cs)], idx_v)
                pltpu.async_copy(table_hbm.at[idx_v], rows_v, sem).wait()
                pltpu.sync_copy(rows_v, out_hbm.at[pl.ds(base, cs)])

    return gk


def _conv_phi_batch(hup_b, wm_ref, bias_ref, acc_scr):
    acc_scr[...] = jnp.zeros((HW, CVAE), jnp.float32)
    t = 0
    for dy in (-1, 0, 1):
        for dx in (-1, 0, 1):
            s = dy * W + dx
            out0 = max(0, -s)
            in0 = max(0, s)
            m = HW - abs(s)
            x = hup_b[in0:in0 + m, :]
            r = lax.broadcasted_iota(jnp.int32, (m, 1), 0) + out0
            ok = None
            if dx != 0:
                wcol = lax.rem(r, W)
                ok = wcol >= 1 if dx == -1 else wcol <= W - 2
            if dy != 0:
                hrow = lax.div(r, W)
                c = hrow >= 1 if dy == -1 else hrow <= H - 2
                ok = c if ok is None else jnp.logical_and(ok, c)
            if ok is not None:
                x = x * jnp.where(ok, 1.0, 0.0).astype(jnp.float32)
            wmat = wm_ref[pl.ds(t * CVAE, CVAE), :]
            contrib = lax.dot_general(x, wmat, (((1,), (0,)), ((), ())),
                                      precision=PREC_CONV,
                                      preferred_element_type=jnp.float32)
            acc_scr[out0:out0 + m, :] = acc_scr[out0:out0 + m, :] + contrib
            t += 1
    return hup_b * (1.0 - RESI) + (acc_scr[...] + bias_ref[...]) * RESI


def _update_body(si, h_ref, frest_ref, wm_ref, bias_ref, u2_ref, ftok_ref,
                 frest_out, fhat_out, mse_ref, acc_scr):
    pn = PNS[si]
    last = si == len(PNS) - 1
    msum = jnp.zeros((1, 1), jnp.float32)
    for b in range(B):
        if not last:
            hup_b = lax.dot_general(
                u2_ref[...], h_ref[pl.ds(b * pn * pn, pn * pn), :CVAE],
                (((1,), (0,)), ((), ())), precision=PREC_LIN,
                preferred_element_type=jnp.float32)
        else:
            hup_b = h_ref[pl.ds(b * HW, HW), :CVAE]
        phi = _conv_phi_batch(hup_b, wm_ref, bias_ref, acc_scr)
        fr = frest_ref[pl.ds(b * HW, HW), :] - phi
        if frest_out is not None:
            frest_out[pl.ds(b * HW, HW), :] = fr
        if fhat_out is not None:
            fhat_out[pl.ds(b * HW, HW), :] = ftok_ref[pl.ds(b * HW, HW), :] - fr
        msum = msum + jnp.sum(fr * fr, keepdims=True)
    mse_ref[...] = msum * (1.0 / (NTOK * CVAE))


def _make_update_call(si):
    pn = PNS[si]
    last = si == len(PNS) - 1
    scratch = [pltpu.VMEM((HW, CVAE), jnp.float32)]
    if not last:
        out_shape = [jax.ShapeDtypeStruct((NTOK, CVAE), jnp.float32),
                     jax.ShapeDtypeStruct((1, 1), jnp.float32)]
        u2 = jnp.asarray(UP2[pn])

        def body(h_ref, frest_ref, wm_ref, bias_ref, u2_ref, frest_out,
                 mse_ref, acc_scr):
            _update_body(si, h_ref, frest_ref, wm_ref, bias_ref, u2_ref,
                         None, frest_out, None, mse_ref, acc_scr)
        call = pl.pallas_call(body, out_shape=out_shape,
                              scratch_shapes=scratch)
        return lambda h, frest, wm, bias, ftok: call(h, frest, wm, bias, u2)
    else:
        out_shape = [jax.ShapeDtypeStruct((NTOK, CVAE), jnp.float32),
                     jax.ShapeDtypeStruct((1, 1), jnp.float32)]

        def body5(h_ref, frest_ref, wm_ref, bias_ref, ftok_ref, fhat_out,
                  mse_ref, acc_scr):
            _update_body(si, h_ref, frest_ref, wm_ref, bias_ref, None,
                         ftok_ref, None, fhat_out, mse_ref, acc_scr)
        call = pl.pallas_call(body5, out_shape=out_shape,
                              scratch_shapes=scratch)
        return lambda h, frest, wm, bias, ftok: call(h, frest, wm, bias, ftok)


_ARGMAX = [_make_argmax_call(si) for si in range(len(PNS))]
_GATHER = functools.cache(lambda si: _make_gather_call(B * PNS[si] * PNS[si]))
_UPDATE = [_make_update_call(si) for si in range(len(PNS))]


def kernel(f_BChw, embedding, phi_w, phi_b):
    ftok = f_BChw.reshape(B, CVAE, HW).transpose(0, 2, 1).reshape(NTOK, CVAE)
    wm_all = phi_w.transpose(0, 3, 4, 2, 1).reshape(NPHI, 9 * CVAE, CVAE)
    bias_all = phi_b.reshape(NPHI, 1, CVAE)

    emb128 = jnp.pad(embedding, ((0, 0), (0, 128 - CVAE)))

    frest = ftok
    fhat = None
    idx_list = []
    mse_sum = jnp.zeros((), jnp.float32)
    for si, pn in enumerate(PNS):
        idx2 = _ARGMAX[si](frest, embedding)
        idx = idx2.reshape(B * pn * pn)
        idx_list.append(idx)
        h = _GATHER(si)(emb128, idx)
        k = K_MAP[si]
        state, mse = _UPDATE[si](h, frest, wm_all[k], bias_all[k], ftok)
        if si != len(PNS) - 1:
            frest = state
        else:
            fhat = state
        mse_sum = mse_sum + mse.reshape(())

    loss = (BETA * mse_sum + mse_sum) / len(PNS)
    fhat_st = fhat.reshape(B, HW, CVAE).transpose(0, 2, 1).reshape(
        B, CVAE, H, W)
    return fhat_st, loss, tuple(idx_list)
